# baseline (speedup 1.0000x reference)
import functools

import jax
import jax.numpy as jnp
from jax import lax
from jax.experimental import pallas as pl
from jax.experimental.pallas import tpu as pltpu
from jax.experimental.pallas import tpu_sc as plsc

N = 10000
E = 320000
NG = 64
D = 128
BN_EPS = 1e-5

NC = 2
NS = 16
NW = NC * NS
CH = 128
GCH = 256
EPAD = 327680
EPW = EPAD // NW
NCHG = EPW // GCH
NCH = EPW // CH
NPAD = 10240
RPS = NPAD // NS
ND = 4


def _sc_aggregate(h, src3, dst3, zeros):
    mesh = plsc.VectorSubcoreMesh(
        core_axis_name="c", subcore_axis_name="s", num_cores=NC, num_subcores=NS
    )

    @functools.partial(
        pl.kernel,
        out_type=jax.ShapeDtypeStruct((NC, NPAD, D), jnp.float32),
        mesh=mesh,
        scratch_types=[
            pltpu.VMEM((EPW,), jnp.int32),
            pltpu.VMEM((ND, 1, CH), jnp.int32),
            pltpu.VMEM((GCH, D), jnp.float32),
            pltpu.VMEM_SHARED((NPAD, D), jnp.float32),
        ]
        + [pltpu.SemaphoreType.DMA] * (2 + ND),
    )
    def agg_kernel(h_hbm, src_hbm, dst_hbm, z_hbm, out_hbm,
                   sidx, didx, rows, acc, *sems):
        ssem = sems[:2]
        dsem = sems[2:]
        c = lax.axis_index("c")
        s = lax.axis_index("s")
        w = c * NS + s

        pltpu.sync_copy(src_hbm.at[pl.ds(w * EPW, EPW)], sidx)
        pltpu.sync_copy(z_hbm.at[pl.ds(s * RPS, RPS)],
                        acc.at[pl.ds(s * RPS, RPS)])
        plsc.subcore_barrier()

        def fire_didx(q, b):
            pltpu.async_copy(dst_hbm.at[w * NCH + q], didx.at[b], dsem[b])

        def wait_didx(b):
            pltpu.make_async_copy(dst_hbm.at[0], didx.at[b], dsem[b]).wait()

        def gather(j):
            off = pl.multiple_of(j * GCH, GCH)
            pltpu.sync_copy(h_hbm.at[sidx.at[pl.ds(off, GCH)]], rows)

        def fire_scatter(b, half):
            pltpu.async_copy(rows.at[pl.ds(half * CH, CH)],
                             acc.at[didx.at[b].at[0]], ssem[half],
                             add=True)

        def wait_scatter(half):
            pltpu.make_async_copy(rows.at[pl.ds(half * CH, CH)],
                                  acc.at[didx.at[0].at[0]], ssem[half]).wait()

        def chunk(j, par, drain, prefetch):
            mine = (2 * par) % ND
            nxt = (2 * par + 2) % ND
            if drain:
                wait_scatter(0)
                wait_scatter(1)
            if prefetch:
                fire_didx(2 * j + 2, nxt)
                fire_didx(2 * j + 3, nxt + 1)
            gather(j)
            for half in (0, 1):
                wait_didx(mine + half)
                fire_scatter(mine + half, half)

        fire_didx(0, 0)
        fire_didx(1, 1)
        chunk(0, 0, drain=False, prefetch=True)

        @pl.loop(1, NCHG - 1, step=2)
        def _(g):
            chunk(g, 1, drain=True, prefetch=True)
            chunk(g + 1, 0, drain=True, prefetch=True)

        chunk(NCHG - 1, 1, drain=True, prefetch=False)
        wait_scatter(0)
        wait_scatter(1)

        plsc.subcore_barrier()
        pltpu.sync_copy(acc.at[pl.ds(s * RPS, RPS)],
                        out_hbm.at[c].at[pl.ds(s * RPS, RPS)])

    return agg_kernel(h, src3, dst3, zeros)


def _tc_layer_body(h_ref, p_ref, w1_ref, b1_ref, w2_ref, b2_ref,
                   g_ref, be_ref, o_ref):
    hs = h_ref[...] + p_ref[0, :N, :] + p_ref[1, :N, :]
    a = jnp.maximum(
        jnp.dot(hs, w1_ref[...], preferred_element_type=jnp.float32)
        + b1_ref[...], 0.0)
    h2 = (jnp.dot(a, w2_ref[...], preferred_element_type=jnp.float32)
          + b2_ref[...])
    m = jnp.mean(h2, axis=0, keepdims=True)
    v = jnp.mean((h2 - m) * (h2 - m), axis=0, keepdims=True)
    o_ref[...] = jnp.maximum(
        (h2 - m) * jax.lax.rsqrt(v + BN_EPS) * g_ref[...] + be_ref[...], 0.0)


def _tc_layer(h, p, W1, b1, W2, b2, g, be):
    return pl.pallas_call(
        _tc_layer_body,
        out_shape=jax.ShapeDtypeStruct((N, D), jnp.float32),
    )(h, p, W1, b1, W2, b2, g, be)


def _tc_layer_pool_body(h_ref, p_ref, w1_ref, b1_ref, w2_ref, b2_ref,
                        g_ref, be_ref, batch_ref, o_ref):
    hs = h_ref[...] + p_ref[0, :N, :] + p_ref[1, :N, :]
    a = jnp.maximum(
        jnp.dot(hs, w1_ref[...], preferred_element_type=jnp.float32)
        + b1_ref[...], 0.0)
    h2 = (jnp.dot(a, w2_ref[...], preferred_element_type=jnp.float32)
          + b2_ref[...])
    m = jnp.mean(h2, axis=0, keepdims=True)
    v = jnp.mean((h2 - m) * (h2 - m), axis=0, keepdims=True)
    hf = jnp.maximum(
        (h2 - m) * jax.lax.rsqrt(v + BN_EPS) * g_ref[...] + be_ref[...], 0.0)
    gids = lax.broadcasted_iota(jnp.int32, (N, NG), 1)
    onehot = (batch_ref[...] == gids).astype(jnp.float32)
    sums = lax.dot_general(onehot, hf, (((0,), (0,)), ((), ())),
                           preferred_element_type=jnp.float32)
    cnt = lax.dot_general(onehot, jnp.ones((N, 1), jnp.float32),
                          (((0,), (0,)), ((), ())),
                          preferred_element_type=jnp.float32)
    o_ref[...] = sums / jnp.clip(cnt, 1.0, None)


def _tc_layer_pool(h, p, W1, b1, W2, b2, g, be, batch):
    return pl.pallas_call(
        _tc_layer_pool_body,
        out_shape=jax.ShapeDtypeStruct((NG, D), jnp.float32),
    )(h, p, W1, b1, W2, b2, g, be, batch)


def kernel(x, edge_index, batch,
           W1_0, b1_0, W2_0, b2_0, g_0, be_0,
           W1_1, b1_1, W2_1, b2_1, g_1, be_1,
           W1_2, b1_2, W2_2, b2_2, g_2, be_2):
    pad = EPAD - E
    src1 = jnp.concatenate([edge_index[0], jnp.zeros((pad,), jnp.int32)])
    dst3 = jnp.concatenate(
        [edge_index[1], jnp.full((pad,), N, jnp.int32)]).reshape(NW * NCH, 1, CH)
    zeros = jnp.zeros((NPAD, D), jnp.float32)
    batch2d = batch.reshape(N, 1)
    params = [(W1_0, b1_0, W2_0, b2_0, g_0, be_0),
              (W1_1, b1_1, W2_1, b2_1, g_1, be_1),
              (W1_2, b1_2, W2_2, b2_2, g_2, be_2)]

    h = x
    for i, (W1, b1, W2, b2, g, be) in enumerate(params):
        p = _sc_aggregate(h, src1, dst3, zeros)
        b1r = b1.reshape(1, D)
        b2r = b2.reshape(1, D)
        gr = g.reshape(1, D)
        ber = be.reshape(1, D)
        if i < 2:
            h = _tc_layer(h, p, W1, b1r, W2, b2r, gr, ber)
        else:
            h = _tc_layer_pool(h, p, W1, b1r, W2, b2r, gr, ber, batch2d)
    return h


# speedup vs baseline: 2.1757x; 2.1757x over previous
import functools

import jax
import jax.numpy as jnp
from jax import lax
from jax.experimental import pallas as pl
from jax.experimental.pallas import tpu as pltpu
from jax.experimental.pallas import tpu_sc as plsc

N = 10000
E = 320000
NG = 64
D = 128
BN_EPS = 1e-5

NC = 2
NS = 16
NW = NC * NS
CH = 128
EPW = E // NW
N_FULL = EPW // CH
TAIL = EPW - N_FULL * CH
NPAD = 10240
RPS = NPAD // NS


def _sc_aggregate(h, src, dst, zeros):
    mesh = plsc.VectorSubcoreMesh(
        core_axis_name="c", subcore_axis_name="s", num_cores=NC, num_subcores=NS
    )

    @functools.partial(
        pl.kernel,
        out_type=jax.ShapeDtypeStruct((NC, NPAD, D), jnp.float32),
        mesh=mesh,
        scratch_types=[
            pltpu.VMEM((2, CH), jnp.int32),
            pltpu.VMEM((2, CH), jnp.int32),
            pltpu.VMEM((CH, D), jnp.float32),
            pltpu.VMEM((2, TAIL), jnp.int32),
            pltpu.VMEM((2, TAIL), jnp.int32),
            pltpu.VMEM((TAIL, D), jnp.float32),
            pltpu.VMEM_SHARED((NPAD, D), jnp.float32),
        ],
    )
    def agg_kernel(h_hbm, src_hbm, dst_hbm, z_hbm, out_hbm,
                   sidx, didx, rows, sidx_t, didx_t, rows_t, acc):
        c = lax.axis_index("c")
        s = lax.axis_index("s")
        pltpu.sync_copy(z_hbm.at[pl.ds(s * RPS, RPS)],
                        acc.at[pl.ds(s * RPS, RPS)])
        plsc.subcore_barrier()

        base = (c * NS + s) * EPW

        @pl.loop(0, N_FULL)
        def _(i):
            off = base + i * CH
            pltpu.sync_copy(src_hbm.at[pl.ds(off, CH)], sidx.at[0])
            pltpu.sync_copy(dst_hbm.at[pl.ds(off, CH)], didx.at[0])
            pltpu.sync_copy(h_hbm.at[sidx.at[0]], rows)
            pltpu.sync_copy(rows, acc.at[didx.at[0]], add=True)

        t_off = base + N_FULL * CH
        pltpu.sync_copy(src_hbm.at[pl.ds(t_off, TAIL)], sidx_t.at[0])
        pltpu.sync_copy(dst_hbm.at[pl.ds(t_off, TAIL)], didx_t.at[0])
        pltpu.sync_copy(h_hbm.at[sidx_t.at[0]], rows_t)
        pltpu.sync_copy(rows_t, acc.at[didx_t.at[0]], add=True)

        plsc.subcore_barrier()
        pltpu.sync_copy(acc.at[pl.ds(s * RPS, RPS)],
                        out_hbm.at[c].at[pl.ds(s * RPS, RPS)])

    return agg_kernel(h, src, dst, zeros)


def _tc_layer_body(h_ref, p_ref, w1_ref, b1_ref, w2_ref, b2_ref,
                   g_ref, be_ref, o_ref):
    hs = h_ref[...] + p_ref[0, :N, :] + p_ref[1, :N, :]
    a = jnp.maximum(
        jnp.dot(hs, w1_ref[...], preferred_element_type=jnp.float32)
        + b1_ref[...], 0.0)
    h2 = (jnp.dot(a, w2_ref[...], preferred_element_type=jnp.float32)
          + b2_ref[...])
    m = jnp.mean(h2, axis=0, keepdims=True)
    v = jnp.mean((h2 - m) * (h2 - m), axis=0, keepdims=True)
    o_ref[...] = jnp.maximum(
        (h2 - m) * jax.lax.rsqrt(v + BN_EPS) * g_ref[...] + be_ref[...], 0.0)


def _tc_layer(h, p, W1, b1, W2, b2, g, be):
    return pl.pallas_call(
        _tc_layer_body,
        out_shape=jax.ShapeDtypeStruct((N, D), jnp.float32),
    )(h, p, W1, b1, W2, b2, g, be)


def _tc_layer_pool_body(h_ref, p_ref, w1_ref, b1_ref, w2_ref, b2_ref,
                        g_ref, be_ref, batch_ref, o_ref):
    hs = h_ref[...] + p_ref[0, :N, :] + p_ref[1, :N, :]
    a = jnp.maximum(
        jnp.dot(hs, w1_ref[...], preferred_element_type=jnp.float32)
        + b1_ref[...], 0.0)
    h2 = (jnp.dot(a, w2_ref[...], preferred_element_type=jnp.float32)
          + b2_ref[...])
    m = jnp.mean(h2, axis=0, keepdims=True)
    v = jnp.mean((h2 - m) * (h2 - m), axis=0, keepdims=True)
    hf = jnp.maximum(
        (h2 - m) * jax.lax.rsqrt(v + BN_EPS) * g_ref[...] + be_ref[...], 0.0)
    gids = lax.broadcasted_iota(jnp.int32, (N, NG), 1)
    onehot = (batch_ref[...] == gids).astype(jnp.float32)
    sums = lax.dot_general(onehot, hf, (((0,), (0,)), ((), ())),
                           preferred_element_type=jnp.float32)
    cnt = lax.dot_general(onehot, jnp.ones((N, 1), jnp.float32),
                          (((0,), (0,)), ((), ())),
                          preferred_element_type=jnp.float32)
    o_ref[...] = sums / jnp.clip(cnt, 1.0, None)


def _tc_layer_pool(h, p, W1, b1, W2, b2, g, be, batch):
    return pl.pallas_call(
        _tc_layer_pool_body,
        out_shape=jax.ShapeDtypeStruct((NG, D), jnp.float32),
    )(h, p, W1, b1, W2, b2, g, be, batch)


def kernel(x, edge_index, batch,
           W1_0, b1_0, W2_0, b2_0, g_0, be_0,
           W1_1, b1_1, W2_1, b2_1, g_1, be_1,
           W1_2, b1_2, W2_2, b2_2, g_2, be_2):
    src = edge_index[0]
    dst = edge_index[1]
    zeros = jnp.zeros((NPAD, D), jnp.float32)
    batch2d = batch.reshape(N, 1)
    params = [(W1_0, b1_0, W2_0, b2_0, g_0, be_0),
              (W1_1, b1_1, W2_1, b2_1, g_1, be_1),
              (W1_2, b1_2, W2_2, b2_2, g_2, be_2)]

    h = x
    for i, (W1, b1, W2, b2, g, be) in enumerate(params):
        p = _sc_aggregate(h, src, dst, zeros)
        b1r = b1.reshape(1, D)
        b2r = b2.reshape(1, D)
        gr = g.reshape(1, D)
        ber = be.reshape(1, D)
        if i < 2:
            h = _tc_layer(h, p, W1, b1r, W2, b2r, gr, ber)
        else:
            h = _tc_layer_pool(h, p, W1, b1r, W2, b2r, gr, ber, batch2d)
    return h
